# baseline (speedup 1.0000x reference)
import functools

import jax
import jax.numpy as jnp
import numpy as np
from jax import lax
from jax.experimental import pallas as pl
from jax.experimental.pallas import tpu as pltpu
from jax.experimental.pallas import tpu_sc as plsc

_CHUNK = 50
_OVERLAP = 7
_EXT = 64
_D = 128
_B = 16
_QLEN = 30
_L = 2048
_NCH = 41
_HALF = 15360
_WB = 3840
_WGRID = 4
_TBUF = 2112

_mesh = plsc.VectorSubcoreMesh(core_axis_name="c", subcore_axis_name="s", num_cores=1)
_sc_params = pltpu.CompilerParams(needs_layout_passes=False)


def _matmul_body(qids_ref, emb_any, e_lo, e_hi, o_ref, rows, qv, sem):
    i = pl.program_id(0)

    @pl.when(i == 0)
    def _():
        def issue(j, carry):
            for b in range(_B):
                idx = qids_ref[b, j]
                pltpu.make_async_copy(
                    emb_any.at[pl.ds(idx, 1)],
                    rows.at[pl.ds(j * _B + b, 1)], sem).start()
            return carry

        lax.fori_loop(0, _QLEN, issue, 0)
        pltpu.make_async_copy(
            emb_any.at[pl.ds(0, _B * _QLEN)], rows, sem).wait()
        acc = jnp.zeros((_B, _D), jnp.float32)
        for j in range(_QLEN):
            acc = acc + rows[pl.ds(16 * j, 16), :]
        qv[...] = acc * jnp.float32(1.0 / _QLEN)

    @pl.when(i > 0)
    def _():
        dims = (((1,), (1,)), ((), ()))
        s_lo = lax.dot_general(qv[...], e_lo[...], dims,
                               preferred_element_type=jnp.float32)
        s_hi = lax.dot_general(qv[...], e_hi[...], dims,
                               preferred_element_type=jnp.float32)
        lo = lax.bitcast_convert_type(
            s_lo.astype(jnp.bfloat16), jnp.uint16).astype(jnp.int32)
        hi = lax.bitcast_convert_type(
            s_hi.astype(jnp.bfloat16), jnp.uint16).astype(jnp.int32)
        o_ref[...] = jnp.bitwise_or(lo, lax.shift_left(hi, 16))


_scores_call = pl.pallas_call(
    _matmul_body,
    grid=(_WGRID + 1,),
    in_specs=[
        pl.BlockSpec((_B, _QLEN), lambda i: (0, 0), memory_space=pltpu.SMEM),
        pl.BlockSpec(memory_space=pl.ANY),
        pl.BlockSpec((_WB, _D), lambda i: (jnp.maximum(i - 1, 0), 0)),
        pl.BlockSpec((_WB, _D), lambda i: (_WGRID + jnp.maximum(i - 1, 0), 0)),
    ],
    out_specs=pl.BlockSpec((_B, _WB), lambda i: (0, jnp.maximum(i - 1, 0))),
    out_shape=jax.ShapeDtypeStruct((_B, _HALF), jnp.int32),
    scratch_shapes=[
        pltpu.VMEM((_B * _QLEN, _D), jnp.float32),
        pltpu.VMEM((_B, _D), jnp.float32),
        pltpu.SemaphoreType.DMA,
    ],
)


def _score_body(dids_hbm, st_hbm, rtab_hbm, out_hbm, ids_v, srow_v, rtab_v,
                tbuf, nbuf, obuf, sem1, sem2, sem3):
    s = lax.axis_index("s")
    b = s

    @pl.when(s < 16)
    def _():
        cp_ids = pltpu.async_copy(dids_hbm.at[b], ids_v, sem1)
        cp_row = pltpu.async_copy(st_hbm.at[b], srow_v, sem2)
        cp_tab = pltpu.async_copy(rtab_hbm, rtab_v, sem3)
        zero = jnp.zeros((16,), jnp.float32)
        for off in (0, 2048, 2064, 2080, 2096):
            tbuf[pl.ds(off, 16)] = zero
            nbuf[pl.ds(off, 16)] = zero
        cp_ids.wait()
        cp_row.wait()
        cp_tab.wait()

        def body(g2, carry):
            for u in range(2):
                g = 2 * g2 + u
                idx = ids_v[pl.ds(16 * g, 16)]
                m = idx != 0
                m_lo = idx < _HALF
                w = jnp.where(m_lo, idx, idx - _HALF)
                word = plsc.load_gather(srow_v, [w])
                bits = jnp.where(m_lo, lax.shift_left(word, 16),
                                 jnp.bitwise_and(word, jnp.int32(-65536)))
                tv = plsc.bitcast(bits, jnp.float32)
                tbuf[pl.ds(_OVERLAP + 16 * g, 16)] = jnp.where(m, tv, 0.0)
                nbuf[pl.ds(_OVERLAP + 16 * g, 16)] = jnp.where(m, 1.0, 0.0)
            return carry

        lax.fori_loop(0, _L // 32, body, 0)

        center_tail = lax.iota(jnp.int32, 16) < 2
        acc = jnp.float32(-3e38)
        for ci in range(_NCH):
            base = _CHUNK * ci
            tsum = (tbuf[pl.ds(base, 16)] + tbuf[pl.ds(base + 16, 16)]
                    + tbuf[pl.ds(base + 32, 16)] + tbuf[pl.ds(base + 48, 16)])
            ssum = jnp.sum(tsum)
            nsum = (nbuf[pl.ds(base, 16)] + nbuf[pl.ds(base + 16, 16)]
                    + nbuf[pl.ds(base + 32, 16)] + nbuf[pl.ds(base + 48, 16)])
            cnt = jnp.sum(nsum)
            csum = (nbuf[pl.ds(base + 7, 16)] + nbuf[pl.ds(base + 23, 16)]
                    + nbuf[pl.ds(base + 39, 16)]
                    + jnp.where(center_tail, nbuf[pl.ds(base + 55, 16)], 0.0))
            ccnt = jnp.sum(csum)
            val = ssum * rtab_v[pl.ds(cnt.astype(jnp.int32), 16)][0]
            val = jnp.where(ccnt > 0.0, val, 0.0)
            val = jnp.where(val == 0.0, jnp.float32(-9000.0), val)
            acc = jnp.maximum(acc, val)
        obuf[...] = jnp.broadcast_to(acc, (16,))
        pltpu.sync_copy(obuf, out_hbm.at[b])


_score_call = functools.partial(
    pl.kernel,
    out_type=jax.ShapeDtypeStruct((_B, 16), jnp.float32),
    mesh=_mesh,
    scratch_types=[
        pltpu.VMEM((_L,), jnp.int32),
        pltpu.VMEM((_HALF,), jnp.int32),
        pltpu.VMEM((80,), jnp.float32),
        pltpu.VMEM((_TBUF,), jnp.float32),
        pltpu.VMEM((_TBUF,), jnp.float32),
        pltpu.VMEM((16,), jnp.float32),
        pltpu.SemaphoreType.DMA,
        pltpu.SemaphoreType.DMA,
        pltpu.SemaphoreType.DMA,
    ],
    compiler_params=_sc_params,
)(_score_body)

_RECIP_TABLE = np.array(
    [1.0 / max(i, 1) for i in range(_EXT + 1)] + [0.0] * (80 - _EXT - 1),
    dtype=np.float32)


def kernel(query_input_ids, query_attention_mask, document_input_ids, emb):
    del query_attention_mask
    d_ids = document_input_ids[:, 1:]
    s_t = _scores_call(query_input_ids, emb, emb, emb)
    out2 = _score_call(d_ids, s_t, jnp.asarray(_RECIP_TABLE))
    return out2[:, 0]


# speedup vs baseline: 1.0102x; 1.0102x over previous
import functools

import jax
import jax.numpy as jnp
import numpy as np
from jax import lax
from jax.experimental import pallas as pl
from jax.experimental.pallas import tpu as pltpu
from jax.experimental.pallas import tpu_sc as plsc

_CHUNK = 50
_OVERLAP = 7
_EXT = 64
_D = 128
_B = 16
_QLEN = 30
_L = 2048
_NCH = 41
_HALF = 15360
_WB = 7680
_WGRID = 2
_TBUF = 2112

_mesh = plsc.VectorSubcoreMesh(core_axis_name="c", subcore_axis_name="s", num_cores=1)
_sc_params = pltpu.CompilerParams(needs_layout_passes=False)


def _matmul_body(qids_ref, emb_any, e_lo, e_hi, o_ref, rows, qv, sem):
    i = pl.program_id(0)

    @pl.when(i == 0)
    def _():
        def issue(j, carry):
            for b in range(_B):
                idx = qids_ref[b, j]
                pltpu.make_async_copy(
                    emb_any.at[pl.ds(idx, 1)],
                    rows.at[pl.ds(j * _B + b, 1)], sem).start()
            return carry

        lax.fori_loop(0, _QLEN, issue, 0)
        pltpu.make_async_copy(
            emb_any.at[pl.ds(0, _B * _QLEN)], rows, sem).wait()
        acc = jnp.zeros((_B, _D), jnp.float32)
        for j in range(_QLEN):
            acc = acc + rows[pl.ds(16 * j, 16), :]
        qv[...] = acc * jnp.float32(1.0 / _QLEN)

    @pl.when(i > 0)
    def _():
        dims = (((1,), (1,)), ((), ()))
        s_lo = lax.dot_general(qv[...], e_lo[...], dims,
                               preferred_element_type=jnp.float32)
        s_hi = lax.dot_general(qv[...], e_hi[...], dims,
                               preferred_element_type=jnp.float32)
        lo = lax.bitcast_convert_type(
            s_lo.astype(jnp.bfloat16), jnp.uint16).astype(jnp.int32)
        hi = lax.bitcast_convert_type(
            s_hi.astype(jnp.bfloat16), jnp.uint16).astype(jnp.int32)
        o_ref[...] = jnp.bitwise_or(lo, lax.shift_left(hi, 16))


_scores_call = pl.pallas_call(
    _matmul_body,
    grid=(_WGRID + 1,),
    in_specs=[
        pl.BlockSpec((_B, _QLEN), lambda i: (0, 0), memory_space=pltpu.SMEM),
        pl.BlockSpec(memory_space=pl.ANY),
        pl.BlockSpec((_WB, _D), lambda i: (jnp.maximum(i - 1, 0), 0)),
        pl.BlockSpec((_WB, _D), lambda i: (_WGRID + jnp.maximum(i - 1, 0), 0)),
    ],
    out_specs=pl.BlockSpec((_B, _WB), lambda i: (0, jnp.maximum(i - 1, 0))),
    out_shape=jax.ShapeDtypeStruct((_B, _HALF), jnp.int32),
    scratch_shapes=[
        pltpu.VMEM((_B * _QLEN, _D), jnp.float32),
        pltpu.VMEM((_B, _D), jnp.float32),
        pltpu.SemaphoreType.DMA,
    ],
)


def _score_body(dids_hbm, st_hbm, rtab_hbm, out_hbm, ids_v, srow_v, rtab_v,
                tbuf, nbuf, obuf, sem1, sem2, sem3):
    s = lax.axis_index("s")
    b = s

    @pl.when(s < 16)
    def _():
        cp_ids = pltpu.async_copy(dids_hbm.at[b], ids_v, sem1)
        cp_row = pltpu.async_copy(st_hbm.at[b], srow_v, sem2)
        cp_tab = pltpu.async_copy(rtab_hbm, rtab_v, sem3)
        zero = jnp.zeros((16,), jnp.float32)
        for off in (0, 2048, 2064, 2080, 2096):
            tbuf[pl.ds(off, 16)] = zero
            nbuf[pl.ds(off, 16)] = zero
        cp_ids.wait()
        cp_row.wait()
        cp_tab.wait()

        def body(g2, carry):
            for u in range(2):
                g = 2 * g2 + u
                idx = ids_v[pl.ds(16 * g, 16)]
                m = idx != 0
                m_lo = idx < _HALF
                w = jnp.where(m_lo, idx, idx - _HALF)
                word = plsc.load_gather(srow_v, [w])
                bits = jnp.where(m_lo, lax.shift_left(word, 16),
                                 jnp.bitwise_and(word, jnp.int32(-65536)))
                tv = plsc.bitcast(bits, jnp.float32)
                tbuf[pl.ds(_OVERLAP + 16 * g, 16)] = jnp.where(m, tv, 0.0)
                nbuf[pl.ds(_OVERLAP + 16 * g, 16)] = jnp.where(m, 1.0, 0.0)
            return carry

        lax.fori_loop(0, _L // 32, body, 0)

        center_tail = lax.iota(jnp.int32, 16) < 2
        acc = jnp.float32(-3e38)
        for ci in range(_NCH):
            base = _CHUNK * ci
            tsum = (tbuf[pl.ds(base, 16)] + tbuf[pl.ds(base + 16, 16)]
                    + tbuf[pl.ds(base + 32, 16)] + tbuf[pl.ds(base + 48, 16)])
            ssum = jnp.sum(tsum)
            nsum = (nbuf[pl.ds(base, 16)] + nbuf[pl.ds(base + 16, 16)]
                    + nbuf[pl.ds(base + 32, 16)] + nbuf[pl.ds(base + 48, 16)])
            cnt = jnp.sum(nsum)
            csum = (nbuf[pl.ds(base + 7, 16)] + nbuf[pl.ds(base + 23, 16)]
                    + nbuf[pl.ds(base + 39, 16)]
                    + jnp.where(center_tail, nbuf[pl.ds(base + 55, 16)], 0.0))
            ccnt = jnp.sum(csum)
            val = ssum * rtab_v[pl.ds(cnt.astype(jnp.int32), 16)][0]
            val = jnp.where(ccnt > 0.0, val, 0.0)
            val = jnp.where(val == 0.0, jnp.float32(-9000.0), val)
            acc = jnp.maximum(acc, val)
        obuf[...] = jnp.broadcast_to(acc, (16,))
        pltpu.sync_copy(obuf, out_hbm.at[b])


_score_call = functools.partial(
    pl.kernel,
    out_type=jax.ShapeDtypeStruct((_B, 16), jnp.float32),
    mesh=_mesh,
    scratch_types=[
        pltpu.VMEM((_L,), jnp.int32),
        pltpu.VMEM((_HALF,), jnp.int32),
        pltpu.VMEM((80,), jnp.float32),
        pltpu.VMEM((_TBUF,), jnp.float32),
        pltpu.VMEM((_TBUF,), jnp.float32),
        pltpu.VMEM((16,), jnp.float32),
        pltpu.SemaphoreType.DMA,
        pltpu.SemaphoreType.DMA,
        pltpu.SemaphoreType.DMA,
    ],
    compiler_params=_sc_params,
)(_score_body)

_RECIP_TABLE = np.array(
    [1.0 / max(i, 1) for i in range(_EXT + 1)] + [0.0] * (80 - _EXT - 1),
    dtype=np.float32)


def kernel(query_input_ids, query_attention_mask, document_input_ids, emb):
    del query_attention_mask
    d_ids = document_input_ids[:, 1:]
    s_t = _scores_call(query_input_ids, emb, emb, emb)
    out2 = _score_call(d_ids, s_t, jnp.asarray(_RECIP_TABLE))
    return out2[:, 0]
